# speedup vs baseline: 3.9289x; 3.9289x over previous
import jax
import jax.numpy as jnp
from jax.experimental import pallas as pl

N_OUT = 64


def _topn_tc_kernel(x_ref, o_ref):
    x = x_ref[...]
    R, C = x.shape
    lane_iota = jax.lax.broadcasted_iota(jnp.int32, (R, C), 1)
    out_iota = jax.lax.broadcasted_iota(jnp.int32, (R, N_OUT), 1)

    def body(i, carry):
        x, out = carry
        m = jnp.max(x, axis=1, keepdims=True)
        eq = x == m
        cand = jnp.where(eq, lane_iota, C)
        j = jnp.min(cand, axis=1, keepdims=True)
        x = jnp.where(lane_iota == j, -jnp.inf, x)
        out = jnp.where(out_iota == i, m, out)
        return x, out

    out0 = jnp.full((R, N_OUT), -jnp.inf, x.dtype)
    _, out = jax.lax.fori_loop(0, N_OUT, body, (x, out0))
    o_ref[...] = out


def kernel(inputs):
    return pl.pallas_call(
        _topn_tc_kernel,
        out_shape=jax.ShapeDtypeStruct((inputs.shape[0], N_OUT), inputs.dtype),
    )(inputs)


# speedup vs baseline: 11.3101x; 2.8787x over previous
import functools

import jax
import jax.numpy as jnp
from jax import lax
from jax.experimental import pallas as pl
from jax.experimental.pallas import tpu as pltpu
from jax.experimental.pallas import tpu_sc as plsc

N_OUT = 64
ROWS = 64
COLS = 8192
LANES = 16
CHUNK = 64
N_CHUNKS = COLS // CHUNK
N_WORKERS = 32
ROWS_PER_WORKER = ROWS // N_WORKERS


def _vsort(v):
    k, _ = plsc.sort_key_val(v, v)
    return k


def _rev(v):
    return lax.rev(v, dimensions=(0,))


def _merge2(a, b):
    rb = _rev(b)
    lo = jnp.minimum(a, rb)
    hi = jnp.maximum(a, rb)
    return _vsort(lo), _vsort(hi)


def _merge4(a0, a1, b0, b1):
    rb0, rb1 = _rev(b1), _rev(b0)
    lo0 = jnp.minimum(a0, rb0)
    lo1 = jnp.minimum(a1, rb1)
    hi0 = jnp.maximum(a0, rb0)
    hi1 = jnp.maximum(a1, rb1)
    l0 = jnp.minimum(lo0, lo1)
    l1 = jnp.maximum(lo0, lo1)
    h0 = jnp.minimum(hi0, hi1)
    h1 = jnp.maximum(hi0, hi1)
    return _vsort(l0), _vsort(l1), _vsort(h0), _vsort(h1)


def _sort64(c0, c1, c2, c3):
    a0, a1 = _merge2(_vsort(c0), _vsort(c1))
    b0, b1 = _merge2(_vsort(c2), _vsort(c3))
    return _merge4(a0, a1, b0, b1)


def _keep_top64(r, c):
    r0, r1, r2, r3 = r
    c0, c1, c2, c3 = c
    rc0, rc1, rc2, rc3 = _rev(c3), _rev(c2), _rev(c1), _rev(c0)
    hi0 = jnp.maximum(r0, rc0)
    hi1 = jnp.maximum(r1, rc1)
    hi2 = jnp.maximum(r2, rc2)
    hi3 = jnp.maximum(r3, rc3)
    l0 = jnp.minimum(hi0, hi2)
    l1 = jnp.minimum(hi1, hi3)
    u0 = jnp.maximum(hi0, hi2)
    u1 = jnp.maximum(hi1, hi3)
    p0 = jnp.minimum(l0, l1)
    p1 = jnp.maximum(l0, l1)
    q0 = jnp.minimum(u0, u1)
    q1 = jnp.maximum(u0, u1)
    return _vsort(p0), _vsort(p1), _vsort(q0), _vsort(q1)


def _row_top64(row_v):
    def load4(base):
        return [row_v[pl.ds(base + k * LANES, LANES)] for k in range(4)]

    r = _sort64(*load4(0))

    def body(i, r):
        c = load4(i * CHUNK)
        return _keep_top64(r, _sort64(*c))

    return lax.fori_loop(1, N_CHUNKS, body, r)


@functools.partial(
    pl.kernel,
    out_type=jax.ShapeDtypeStruct((ROWS, N_OUT), jnp.float32),
    mesh=plsc.VectorSubcoreMesh(core_axis_name="c", subcore_axis_name="s"),
    scratch_types=[
        pltpu.VMEM((COLS,), jnp.float32),
        pltpu.VMEM((N_OUT,), jnp.float32),
    ],
    compiler_params=pltpu.CompilerParams(needs_layout_passes=False),
)
def _sc_topn(x_hbm, o_hbm, row_v, out_v):
    wid = lax.axis_index("s") * 2 + lax.axis_index("c")
    for rr in range(ROWS_PER_WORKER):
        row = wid * ROWS_PER_WORKER + rr
        pltpu.sync_copy(x_hbm.at[row], row_v)
        r0, r1, r2, r3 = _row_top64(row_v)
        out_v[pl.ds(0, LANES)] = _rev(r3)
        out_v[pl.ds(16, LANES)] = _rev(r2)
        out_v[pl.ds(32, LANES)] = _rev(r1)
        out_v[pl.ds(48, LANES)] = _rev(r0)
        pltpu.sync_copy(out_v, o_hbm.at[row])


def kernel(inputs):
    return _sc_topn(inputs)
